# Initial kernel scaffold; baseline (speedup 1.0000x reference)
import jax
import jax.numpy as jnp
from jax.experimental import pallas as pl


def kernel(h, x, mask_ligand, batch, params):
    raise NotImplementedError("write your pallas kernel here")


# speedup vs baseline: 12.8025x; 12.8025x over previous
import functools

import numpy as np
import jax
import jax.numpy as jnp
from jax import lax
from jax.experimental import pallas as pl
from jax.experimental.pallas import tpu as pltpu
from jax.experimental.pallas import tpu_sc as plsc

N_NODES = 10000
HID = 128
NH = 16
K = 32
NG = 20
R_MAX = 10.0
N_GRAPHS = 16
HEAD_DIM = HID // NH

N_PAD = 10240
E_PAD = N_PAD * K
BR = 256
NB = N_PAD // BR
W = 2048
BRF = 128
NBF = N_PAD // BRF
EBF = BRF * K
TD = 256

COEFF = -0.5 / (R_MAX / (NG - 1)) ** 2
INV_SQRT_HD = 1.0 / float(np.sqrt(HEAD_DIM))


def _knn_body(cs_ref, xrows_ref, bcol_ref, xT_ref, bT_ref, out_ref):
    b = pl.program_id(0)
    cs = cs_ref[b]
    xw = xrows_ref[pl.ds(cs, W), :]
    bw = bcol_ref[pl.ds(cs, W), :]
    xb = xT_ref[...]
    bb = bT_ref[...]

    d2 = jnp.zeros((W, BR), jnp.float32)
    for c in range(3):
        diff = xw[:, c:c + 1] - xb[c:c + 1, :]
        d2 = d2 + diff * diff

    ii = lax.broadcasted_iota(jnp.int32, (W, BR), 0)
    gcol = cs + ii
    grow = b * BR + lax.broadcasted_iota(jnp.int32, (W, BR), 1)
    invalid = (bw != bb) | (gcol == grow)
    d2 = jnp.where(invalid, jnp.inf, d2)

    for kk in range(K):
        mval = jnp.min(d2, axis=0, keepdims=True)
        cand = jnp.where(d2 == mval, ii, W)
        idx = jnp.min(cand, axis=0, keepdims=True)
        out_ref[kk:kk + 1, :] = cs + idx
        d2 = jnp.where(ii == idx, jnp.inf, d2)


def _knn(xp, bpf, cs_arr):
    grid_spec = pltpu.PrefetchScalarGridSpec(
        num_scalar_prefetch=1,
        grid=(NB,),
        in_specs=[
            pl.BlockSpec((N_PAD, 3), lambda b, c: (0, 0)),
            pl.BlockSpec((N_PAD, 1), lambda b, c: (0, 0)),
            pl.BlockSpec((3, BR), lambda b, c: (0, b)),
            pl.BlockSpec((1, BR), lambda b, c: (0, b)),
        ],
        out_specs=pl.BlockSpec((K, BR), lambda b, c: (0, b)),
    )
    out = pl.pallas_call(
        _knn_body,
        grid_spec=grid_spec,
        out_shape=jax.ShapeDtypeStruct((K, N_PAD), jnp.int32),
    )(cs_arr, xp, bpf[:, None], xp.T, bpf[None, :])
    return out


_SC_CHUNK = 80


def _sc_gather(table, idx):
    D = table.shape[1]
    info = plsc.get_sparse_core_info()
    nc, ns = info.num_cores, info.num_subcores
    nw = nc * ns
    per_w = E_PAD // nw
    iters = per_w // _SC_CHUNK
    mesh = plsc.VectorSubcoreMesh(core_axis_name="c", subcore_axis_name="s")

    @functools.partial(
        pl.kernel,
        mesh=mesh,
        out_type=jax.ShapeDtypeStruct((E_PAD, D), jnp.float32),
        scratch_types=[
            pltpu.VMEM((_SC_CHUNK,), jnp.int32),
            pltpu.VMEM((_SC_CHUNK, D), jnp.float32),
            pltpu.SemaphoreType.DMA,
        ],
    )
    def gk(table_hbm, idx_hbm, out_hbm, idx_v, rows_v, sem):
        wid = lax.axis_index("s") * nc + lax.axis_index("c")
        base = wid * per_w

        def body(i, carry):
            off = base + i * _SC_CHUNK
            pltpu.sync_copy(idx_hbm.at[pl.ds(off, _SC_CHUNK)], idx_v)
            pltpu.async_copy(table_hbm.at[idx_v], rows_v, sem).wait()
            pltpu.sync_copy(rows_v, out_hbm.at[pl.ds(off, _SC_CHUNK)])
            return carry

        lax.fori_loop(0, iters, body, 0)

    return gk(table, idx)


def _dot(a, b):
    return lax.dot_general(a, b, (((1,), (0,)), ((), ())),
                           preferred_element_type=jnp.float32,
                           precision=lax.Precision.HIGHEST)


def _ln(y, g, be):
    mu = jnp.mean(y, axis=-1, keepdims=True)
    d = y - mu
    var = jnp.mean(d * d, axis=-1, keepdims=True)
    return d * lax.rsqrt(var + 1e-5) * g + be


def _mlp2(x, w1, b1, g, be, w2, b2):
    y = _dot(x, w1) + b1
    y = jnp.maximum(_ln(y, g, be), 0.0)
    return _dot(y, w2) + b2


def _rep(a, reps):
    b, d = a.shape
    return jnp.broadcast_to(a[:, None, :], (b, reps, d)).reshape(b * reps, d)


def _edge_kv_mlp(sm, masks, hj, hproj_rep, wr):
    w_et, w_r, w_hj, b1, g, be, w2, b2 = wr
    y = hproj_rep + _dot(hj, w_hj) + b1
    for t in range(4):
        y = y + masks[t] * (_dot(sm, w_r[NG * t:NG * (t + 1), :]) + w_et[t:t + 1, :])
    y = jnp.maximum(_ln(y, g, be), 0.0)
    return _dot(y, w2) + b2


def _edge_common(x_blk, lig_blk, xj, ligj, off_row, ep):
    xi = _rep(x_blk, K)
    ligi = _rep(lig_blk, K)
    rel = xi - xj
    dist = jnp.sqrt(jnp.sum(rel * rel, axis=-1, keepdims=True))
    dlt = dist - off_row
    sm = jnp.exp(COEFF * (dlt * dlt))
    ew = _mlp2(sm, *ep)
    ew = 1.0 / (1.0 + jnp.exp(-ew))
    et = (1.0 - ligj) * 2.0 + (1.0 - ligi)
    masks = [(et == float(t)).astype(jnp.float32) for t in range(4)]
    return rel, sm, ew, masks


def _head_mats():
    rows = lax.broadcasted_iota(jnp.int32, (HID, NH), 0)
    cols = lax.broadcasted_iota(jnp.int32, (HID, NH), 1)
    s = (rows // HEAD_DIM == cols).astype(jnp.float32)
    return s, s.T


def _softmax_k(s16, brf):
    s3 = s16.reshape(brf, K, NH)
    mx = jnp.max(s3, axis=1, keepdims=True)
    e3 = jnp.exp(s3 - mx)
    den = jnp.sum(e3, axis=1, keepdims=True)
    return (e3 / (den + 1e-16)).reshape(brf * K, NH)


def _x2h_body(h_ref, x_ref, lig_ref, g_ref, off_ref, *rest):
    wr = rest[:-1]
    o_ref = rest[-1]
    ep, whi_k, kv_k = wr[0:6], wr[6], wr[7:15]
    whi_v, kv_v = wr[15], wr[16:24]
    hq = wr[24:30]
    w_o, w_h, no_b1, no_g, no_be, no_w2, no_b2 = wr[30:37]

    h_blk = h_ref[...]
    g_all = g_ref[...]
    hj = g_all[:, 0:HID]
    xj = g_all[:, HID:HID + 3]
    ligj = g_all[:, HID + 3:HID + 4]
    rel, sm, ew, masks = _edge_common(
        x_ref[...], lig_ref[...], xj, ligj, off_ref[...],
        [r[...] for r in ep])

    hik = _rep(_dot(h_blk, whi_k[...]), K)
    k_e = _edge_kv_mlp(sm, masks, hj, hik, [r[...] for r in kv_k])
    hiv = _rep(_dot(h_blk, whi_v[...]), K)
    v_e = _edge_kv_mlp(sm, masks, hj, hiv, [r[...] for r in kv_v]) * ew

    q = _mlp2(h_blk, *[r[...] for r in hq])
    s_mat, s_mat_t = _head_mats()
    s16 = _dot(_rep(q, K) * k_e, s_mat) * INV_SQRT_HD
    al = _softmax_k(s16, BRF)
    al128 = _dot(al, s_mat_t)
    out = jnp.sum((al128 * v_e).reshape(BRF, K, HID), axis=1)

    y = _dot(out, w_o[...]) + _dot(h_blk, w_h[...]) + no_b1[...]
    y = jnp.maximum(_ln(y, no_g[...], no_be[...]), 0.0)
    y = _dot(y, no_w2[...]) + no_b2[...]
    o_ref[...] = y + h_blk


def _h2x_body(h_ref, x_ref, lig_ref, g_ref, g2_ref, off_ref, *rest):
    wr = rest[:-1]
    o_ref = rest[-1]
    ep, whi_k, kv_k = wr[0:6], wr[6], wr[7:15]
    whi_v, kv_v = wr[15], wr[16:24]
    xq = wr[24:30]

    h_blk = h_ref[...]
    x_blk = x_ref[...]
    g_all = g_ref[...]
    hj = g2_ref[...]
    xj = g_all[:, HID:HID + 3]
    ligj = g_all[:, HID + 3:HID + 4]
    rel, sm, ew, masks = _edge_common(
        x_blk, lig_ref[...], xj, ligj, off_ref[...],
        [r[...] for r in ep])

    hik = _rep(_dot(h_blk, whi_k[...]), K)
    k_e = _edge_kv_mlp(sm, masks, hj, hik, [r[...] for r in kv_k])
    hiv = _rep(_dot(h_blk, whi_v[...]), K)
    v16 = _edge_kv_mlp(sm, masks, hj, hiv, [r[...] for r in kv_v]) * ew

    q = _mlp2(h_blk, *[r[...] for r in xq])
    s_mat, _ = _head_mats()
    s16 = _dot(_rep(q, K) * k_e, s_mat) * INV_SQRT_HD
    al = _softmax_k(s16, BRF)
    w_e = jnp.sum(al * v16, axis=-1, keepdims=True) * (1.0 / NH)
    dx = jnp.sum((w_e * rel).reshape(BRF, K, 3), axis=1)
    o_ref[...] = x_blk + dx


def _full_spec(w):
    nd = w.ndim
    return pl.BlockSpec(w.shape, (lambda b, _nd=nd: (0,) * _nd))


def _run_fused(body, node_ins, edge_ins, wlist, out_dim):
    in_specs = (
        [pl.BlockSpec((BRF, a.shape[1]), lambda b: (b, 0)) for a in node_ins]
        + [pl.BlockSpec((EBF, a.shape[1]), lambda b: (b, 0)) for a in edge_ins]
        + [_full_spec(w) for w in wlist]
    )
    return pl.pallas_call(
        body,
        grid=(NBF,),
        in_specs=in_specs,
        out_specs=pl.BlockSpec((BRF, out_dim), lambda b: (b, 0)),
        out_shape=jax.ShapeDtypeStruct((N_PAD, out_dim), jnp.float32),
    )(*node_ins, *edge_ins, *wlist)


def _prep_plain(p):
    return [p['w1'], p['b1'][None, :], p['g'][None, :], p['be'][None, :],
            p['w2'], p['b2'][None, :]]


def _prep_kv(p):
    w1 = p['w1']
    whi = w1[4 + 4 * NG:4 + 4 * NG + HID]
    kv = [w1[0:4], w1[4:4 + 4 * NG], w1[4 + 4 * NG + HID:],
          p['b1'][None, :], p['g'][None, :], p['be'][None, :],
          p['w2'], p['b2'][None, :]]
    return whi, kv


def kernel(h, x, mask_ligand, batch, params):
    pad_n = N_PAD - N_NODES
    hp = jnp.pad(h, ((0, pad_n), (0, 0)))
    xp = jnp.pad(x, ((0, pad_n), (0, 0)))
    ligp = jnp.pad(mask_ligand.astype(jnp.float32), (0, pad_n))[:, None]
    bp = jnp.pad(batch.astype(jnp.int32), (0, pad_n), constant_values=-1)
    bpf = bp.astype(jnp.float32)

    starts = jnp.searchsorted(batch, jnp.arange(N_GRAPHS, dtype=batch.dtype),
                              side='left').astype(jnp.int32)
    g0 = jnp.clip(bp[::BR], 0, N_GRAPHS - 1)
    cs_arr = jnp.minimum((starts[g0] // 256) * 256, N_PAD - W).astype(jnp.int32)

    src2d = _knn(xp, bpf, cs_arr)
    src = jnp.transpose(src2d).reshape(-1)

    table = jnp.concatenate(
        [hp, xp, ligp, jnp.zeros((N_PAD, TD - HID - 4), jnp.float32)], axis=1)
    g_rows = _sc_gather(table, src)

    off_row = jnp.asarray(np.linspace(0.0, R_MAX, NG, dtype=np.float32))[None, :]

    lp = params['layer_0']
    ep_w = _prep_plain(params['edge_pred'])
    whi_k, kv_k = _prep_kv(lp['x2h_0']['hk'])
    whi_v, kv_v = _prep_kv(lp['x2h_0']['hv'])
    hq_w = _prep_plain(lp['x2h_0']['hq'])
    no = lp['x2h_0']['node_out']
    no_w = [no['w1'][0:HID], no['w1'][HID:], no['b1'][None, :],
            no['g'][None, :], no['be'][None, :], no['w2'], no['b2'][None, :]]

    wlist1 = ep_w + [whi_k] + kv_k + [whi_v] + kv_v + hq_w + no_w
    h1p = _run_fused(_x2h_body, [hp, xp, ligp], [g_rows],
                     [off_row] + wlist1, HID)

    g2_rows = _sc_gather(h1p, src)

    whi_xk, kv_xk = _prep_kv(lp['h2x_0']['xk'])
    whi_xv, kv_xv = _prep_kv(lp['h2x_0']['xv'])
    xq_w = _prep_plain(lp['h2x_0']['xq'])
    wlist2 = ep_w + [whi_xk] + kv_xk + [whi_xv] + kv_xv + xq_w
    xnp = _run_fused(_h2x_body, [h1p, xp, ligp], [g_rows, g2_rows],
                     [off_row] + wlist2, 3)

    return h1p[:N_NODES], xnp[:N_NODES]
